# baseline (speedup 1.0000x reference)
import functools

import jax
import jax.numpy as jnp
from jax import lax
from jax.experimental import pallas as pl
from jax.experimental.pallas import tpu as pltpu
from jax.experimental.pallas import tpu_sc as plsc

D = 4
B = 4096
F = 1024
EPS = 1e-3

NC = 2
NS = 16
NW = NC * NS
ROWS = B // NW
CH = 16
NCH = ROWS // CH
LANES = 16
VPR = F // LANES
UNROLL = 8


def _tables_body(gg, gb, dg, db, mm, mv, s_ref, t_ref):
    s = gg[0, 0] * dg[...] * lax.rsqrt(mv[...] + EPS)
    s_ref[...] = s
    t_ref[...] = gb[0, 0] + db[...] - s * mm[...]


def _compute_tables(gg, gb, dg, db, mm, mv):
    return pl.pallas_call(
        _tables_body,
        out_shape=(
            jax.ShapeDtypeStruct((D, F), jnp.float32),
            jax.ShapeDtypeStruct((D, F), jnp.float32),
        ),
    )(gg.reshape(1, 1), gb.reshape(1, 1), dg.reshape(D, 1), db.reshape(D, 1),
      mm, mv)


def _sc_body(x_hbm, ids_hbm, s_hbm, t_hbm, out_hbm,
             ids_v, s_v, t_v, xbuf, obuf):
    wid = lax.axis_index("s") * NC + lax.axis_index("c")
    base = wid * ROWS
    pltpu.sync_copy(ids_hbm.at[pl.ds(base, ROWS)], ids_v)
    pltpu.sync_copy(s_hbm, s_v)
    pltpu.sync_copy(t_hbm, t_v)

    def chunk_body(k, carry):
        r0 = base + k * CH
        pltpu.sync_copy(x_hbm.at[pl.ds(r0, CH)], xbuf)
        dvec = ids_v[pl.ds(k * CH, CH)]

        for i in range(CH):
            d = dvec[i]

            def vec_body(j, carry_v, i=i, d=d):
                for u in range(UNROLL):
                    off = (j * UNROLL + u) * LANES
                    sv = s_v[d, pl.ds(off, LANES)]
                    tv = t_v[d, pl.ds(off, LANES)]
                    xv = xbuf[i, pl.ds(off, LANES)]
                    obuf[i, pl.ds(off, LANES)] = xv * sv + tv
                return carry_v

            lax.fori_loop(0, VPR // UNROLL, vec_body, 0)

        pltpu.sync_copy(obuf, out_hbm.at[pl.ds(r0, CH)])
        return carry

    lax.fori_loop(0, NCH, chunk_body, 0)


@functools.partial(
    pl.kernel,
    out_type=jax.ShapeDtypeStruct((B, F), jnp.float32),
    mesh=plsc.VectorSubcoreMesh(core_axis_name="c", subcore_axis_name="s"),
    scratch_types=[
        pltpu.VMEM((ROWS,), jnp.int32),
        pltpu.VMEM((D, F), jnp.float32),
        pltpu.VMEM((D, F), jnp.float32),
        pltpu.VMEM((CH, F), jnp.float32),
        pltpu.VMEM((CH, F), jnp.float32),
    ],
)
def _sc_apply(x_hbm, ids_hbm, s_hbm, t_hbm, out_hbm,
              ids_v, s_v, t_v, xbuf, obuf):
    _sc_body(x_hbm, ids_hbm, s_hbm, t_hbm, out_hbm,
             ids_v, s_v, t_v, xbuf, obuf)


def kernel(features, domain_types_idx, global_gamma, global_beta,
           domain_gammas, domain_betas, moving_means, moving_vars):
    s_tab, t_tab = _compute_tables(global_gamma, global_beta,
                                   domain_gammas, domain_betas,
                                   moving_means, moving_vars)
    ids = domain_types_idx.reshape(-1)
    return _sc_apply(features, ids, s_tab, t_tab)


# speedup vs baseline: 2.0116x; 2.0116x over previous
import functools

import jax
import jax.numpy as jnp
from jax import lax
from jax.experimental import pallas as pl
from jax.experimental.pallas import tpu as pltpu
from jax.experimental.pallas import tpu_sc as plsc

D = 4
B = 4096
F = 1024
EPS = 1e-3

NC = 2
NS = 16
NW = NC * NS
ROWS = B // NW
CH = 16
NCH = ROWS // CH
LANES = 16
VPR = F // LANES
UNROLL = 8


def _tables_body(gg, gb, dg, db, mm, mv, s_ref, t_ref):
    s = gg[0, 0] * dg[...] * lax.rsqrt(mv[...] + EPS)
    s_ref[...] = s
    t_ref[...] = gb[0, 0] + db[...] - s * mm[...]


def _compute_tables(gg, gb, dg, db, mm, mv):
    return pl.pallas_call(
        _tables_body,
        out_shape=(
            jax.ShapeDtypeStruct((D, F), jnp.float32),
            jax.ShapeDtypeStruct((D, F), jnp.float32),
        ),
    )(gg.reshape(1, 1), gb.reshape(1, 1), dg.reshape(D, 1), db.reshape(D, 1),
      mm, mv)


def _sc_body(x_hbm, ids_hbm, s_hbm, t_hbm, out_hbm,
             ids_v, s_v, t_v, xbuf, obuf,
             in_sems, out_sems):
    wid = lax.axis_index("s") * NC + lax.axis_index("c")
    base = wid * ROWS

    for b in range(2):
        pltpu.async_copy(x_hbm.at[pl.ds(base + b * CH, CH)], xbuf.at[b],
                         in_sems.at[b])
    pltpu.sync_copy(ids_hbm.at[pl.ds(base, ROWS)], ids_v)
    pltpu.sync_copy(s_hbm, s_v)
    pltpu.sync_copy(t_hbm, t_v)

    def _compute_chunk(b, c):
        dvec = ids_v[pl.ds(c * CH, CH)]
        ds = [dvec[i] for i in range(CH)]
        for i in range(CH):
            d = ds[i]

            @plsc.parallel_loop(0, VPR, unroll=UNROLL)
            def vec_body(j, i=i, d=d, b=b):
                off = j * LANES
                sv = s_v[d, pl.ds(off, LANES)]
                tv = t_v[d, pl.ds(off, LANES)]
                xv = xbuf[b, i, pl.ds(off, LANES)]
                obuf[b, i, pl.ds(off, LANES)] = xv * sv + tv

    def round_body(g, carry):
        for b in range(2):
            c = 2 * g + b
            r0 = base + c * CH
            pltpu.make_async_copy(x_hbm.at[pl.ds(r0, CH)], xbuf.at[b],
                                  in_sems.at[b]).wait()

            @pl.when(g > 0)
            def _(b=b, c=c):
                pltpu.make_async_copy(
                    obuf.at[b], out_hbm.at[pl.ds(base + (c - 2) * CH, CH)],
                    out_sems.at[b]).wait()

            _compute_chunk(b, c)
            pltpu.async_copy(obuf.at[b], out_hbm.at[pl.ds(r0, CH)],
                             out_sems.at[b])

            @pl.when(g < NCH // 2 - 1)
            def _(b=b, c=c, r0=r0):
                pltpu.async_copy(x_hbm.at[pl.ds(r0 + 2 * CH, CH)],
                                 xbuf.at[b], in_sems.at[b])

        return carry

    lax.fori_loop(0, NCH // 2, round_body, 0)
    for b in range(2):
        pltpu.make_async_copy(
            obuf.at[b], out_hbm.at[pl.ds(base + (NCH - 2 + b) * CH, CH)],
            out_sems.at[b]).wait()


@functools.partial(
    pl.kernel,
    out_type=jax.ShapeDtypeStruct((B, F), jnp.float32),
    mesh=plsc.VectorSubcoreMesh(core_axis_name="c", subcore_axis_name="s"),
    scratch_types=[
        pltpu.VMEM((ROWS,), jnp.int32),
        pltpu.VMEM((D, F), jnp.float32),
        pltpu.VMEM((D, F), jnp.float32),
        pltpu.VMEM((2, CH, F), jnp.float32),
        pltpu.VMEM((2, CH, F), jnp.float32),
        pltpu.SemaphoreType.DMA((2,)),
        pltpu.SemaphoreType.DMA((2,)),
    ],
)
def _sc_apply(x_hbm, ids_hbm, s_hbm, t_hbm, out_hbm,
              ids_v, s_v, t_v, xbuf, obuf, in_sems, out_sems):
    _sc_body(x_hbm, ids_hbm, s_hbm, t_hbm, out_hbm,
             ids_v, s_v, t_v, xbuf, obuf, in_sems, out_sems)


def kernel(features, domain_types_idx, global_gamma, global_beta,
           domain_gammas, domain_betas, moving_means, moving_vars):
    s_tab, t_tab = _compute_tables(global_gamma, global_beta,
                                   domain_gammas, domain_betas,
                                   moving_means, moving_vars)
    ids = domain_types_idx.reshape(-1)
    return _sc_apply(features, ids, s_tab, t_tab)
